# Initial kernel scaffold; baseline (speedup 1.0000x reference)
import jax
import jax.numpy as jnp
from jax.experimental import pallas as pl

ACTION_SIZE = 4
POINT_SIZE = 11
EMBED_DIM = 128
P_PAD = 16

BLOCK_B = 2048


def _tc_body(ew_ref, av_ref, out_ref):
    ew = ew_ref[...]
    av = av_ref[...]
    num = jax.lax.dot_general(
        ew, av, (((1,), (0,)), ((), ())),
        preferred_element_type=jnp.float32)
    n1 = jnp.sqrt(jnp.sum(ew * ew, axis=1, keepdims=True))
    n2 = jnp.sqrt(jnp.sum(av * av, axis=0, keepdims=True))
    sims = num / jnp.maximum(n1 * n2, 1e-8)
    col = jax.lax.broadcasted_iota(jnp.int32, (BLOCK_B, P_PAD), 1)
    sims = jnp.where(col < POINT_SIZE, sims, -jnp.inf)
    rowmax = jnp.max(sims, axis=1, keepdims=True)
    first_p = jnp.min(jnp.where(sims == rowmax, col, P_PAD), axis=1,
                      keepdims=True)
    action = jnp.where(first_p < 4, 0,
             jnp.where(first_p < 9, 1,
             jnp.where(first_p == 9, 2, 3)))
    a4 = jax.lax.broadcasted_iota(jnp.int32, (BLOCK_B, ACTION_SIZE), 1)
    out_ref[...] = (a4 == action).astype(jnp.float32)


def kernel(embedded_words, action_vectors):
    batch = embedded_words.shape[0]
    av = jnp.pad(action_vectors[0], ((0, 0), (0, P_PAD - POINT_SIZE)))
    grid = (batch // BLOCK_B,)
    return pl.pallas_call(
        _tc_body,
        grid=grid,
        in_specs=[
            pl.BlockSpec((BLOCK_B, EMBED_DIM), lambda i: (i, 0)),
            pl.BlockSpec((EMBED_DIM, P_PAD), lambda i: (0, 0)),
        ],
        out_specs=pl.BlockSpec((BLOCK_B, ACTION_SIZE), lambda i: (i, 0)),
        out_shape=jax.ShapeDtypeStruct((batch, ACTION_SIZE), jnp.float32),
    )(embedded_words, av)


# speedup vs baseline: 22.1335x; 22.1335x over previous
import jax
import jax.numpy as jnp
from jax.experimental import pallas as pl

ACTION_SIZE = 4
POINT_SIZE = 11
EMBED_DIM = 128
P_PAD = 16

BLOCK_B = 2048


def _tc_body(ew_ref, av_ref, out_ref):
    ew = ew_ref[...]
    av = av_ref[...]
    num = jax.lax.dot_general(
        ew, av, (((1,), (0,)), ((), ())),
        precision=jax.lax.Precision.HIGHEST,
        preferred_element_type=jnp.float32)
    n1 = jnp.sqrt(jnp.sum(ew * ew, axis=1, keepdims=True))
    n2 = jnp.sqrt(jnp.sum(av * av, axis=0, keepdims=True))
    sims = num / jnp.maximum(n1 * n2, 1e-8)
    col = jax.lax.broadcasted_iota(jnp.int32, (BLOCK_B, P_PAD), 1)
    sims = jnp.where(col < POINT_SIZE, sims, -jnp.inf)
    rowmax = jnp.max(sims, axis=1, keepdims=True)
    first_p = jnp.min(jnp.where(sims == rowmax, col, P_PAD), axis=1,
                      keepdims=True)
    action = jnp.where(first_p < 4, 0,
             jnp.where(first_p < 9, 1,
             jnp.where(first_p == 9, 2, 3)))
    a4 = jax.lax.broadcasted_iota(jnp.int32, (BLOCK_B, ACTION_SIZE), 1)
    out_ref[...] = (a4 == action).astype(jnp.float32)


def kernel(embedded_words, action_vectors):
    batch = embedded_words.shape[0]
    av = jnp.pad(action_vectors[0], ((0, 0), (0, P_PAD - POINT_SIZE)))
    grid = (batch // BLOCK_B,)
    return pl.pallas_call(
        _tc_body,
        grid=grid,
        in_specs=[
            pl.BlockSpec((BLOCK_B, EMBED_DIM), lambda i: (i, 0)),
            pl.BlockSpec((EMBED_DIM, P_PAD), lambda i: (0, 0)),
        ],
        out_specs=pl.BlockSpec((BLOCK_B, ACTION_SIZE), lambda i: (i, 0)),
        out_shape=jax.ShapeDtypeStruct((batch, ACTION_SIZE), jnp.float32),
    )(embedded_words, av)
